# Initial kernel scaffold; baseline (speedup 1.0000x reference)
import jax
import jax.numpy as jnp
from jax.experimental import pallas as pl


def kernel(x, edge_index, edge_weight, kernel, bias, skip_weight):
    raise NotImplementedError("write your pallas kernel here")


# speedup vs baseline: 3.8152x; 3.8152x over previous
import functools

import jax
import jax.numpy as jnp
from jax import lax
from jax.experimental import pallas as pl
from jax.experimental.pallas import tpu as pltpu
from jax.experimental.pallas import tpu_sc as plsc

N = 10000
C = 128
LANES = 16
NC = 2
NS = 16
NW = NC * NS
K = 128
NPAD = 10240
ROWS_PER_TILE = NPAD // NS
OUT_CHUNK = 128


def _bcast_lane(vec, l):
    idx = jnp.full((LANES, 1), l, jnp.int32)
    dnums = lax.GatherDimensionNumbers(
        offset_dims=(), collapsed_slice_dims=(0,), start_index_map=(0,))
    return lax.gather(vec, idx, dnums, slice_sizes=(1,),
                      mode=lax.GatherScatterMode.PROMISE_IN_BOUNDS)


def _zero_rows(rows_v, nrows):
    zv = jnp.zeros((LANES,), jnp.float32)

    def body(r, carry):
        for j in range(C // LANES):
            rows_v[r, pl.ds(j * LANES, LANES)] = zv
        return carry

    lax.fori_loop(0, nrows, body, 0)


def _sc_body(ept, nchunk, h_hbm, src_hbm, dst_hbm, w_hbm, out_hbm,
             src_v, dst_v, w_v, rows_v, acc_sh, sem):
    c = lax.axis_index("c")
    s = lax.axis_index("s")
    wid = c * NS + s

    _zero_rows(rows_v, OUT_CHUNK)
    for t in range(ROWS_PER_TILE // OUT_CHUNK):
        pltpu.sync_copy(rows_v,
                        acc_sh.at[pl.ds(s * ROWS_PER_TILE + t * OUT_CHUNK,
                                        OUT_CHUNK)])
    plsc.subcore_barrier()

    base = wid * ept

    def chunk_body(i, carry):
        off = base + i * K
        pltpu.sync_copy(src_hbm.at[pl.ds(off, K)], src_v)
        pltpu.sync_copy(dst_hbm.at[pl.ds(off, K)], dst_v)
        pltpu.sync_copy(w_hbm.at[pl.ds(off, K)], w_v)
        pltpu.async_copy(h_hbm.at[src_v], rows_v, sem).wait()

        def group_body(g, gcarry):
            wg = w_v[pl.ds(g * LANES, LANES)]
            for l in range(LANES):
                wv = _bcast_lane(wg, l)
                e = g * LANES + l
                for j in range(C // LANES):
                    sl = pl.ds(j * LANES, LANES)
                    rows_v[e, sl] = rows_v[e, sl] * wv
            return gcarry

        lax.fori_loop(0, K // LANES, group_body, 0)

        pltpu.sync_copy(rows_v, acc_sh.at[dst_v], add=True)
        return carry

    lax.fori_loop(0, nchunk, chunk_body, 0)
    plsc.subcore_barrier()

    for t in range(ROWS_PER_TILE // OUT_CHUNK):
        r0 = s * ROWS_PER_TILE + t * OUT_CHUNK
        pltpu.sync_copy(acc_sh.at[pl.ds(r0, OUT_CHUNK)], rows_v)
        pltpu.sync_copy(rows_v, out_hbm.at[pl.ds(c * NPAD + r0, OUT_CHUNK)])


def _make_sc_call(ept, nchunk):
    mesh = plsc.VectorSubcoreMesh(core_axis_name="c", subcore_axis_name="s")
    return pl.kernel(
        functools.partial(_sc_body, ept, nchunk),
        out_type=jax.ShapeDtypeStruct((NC * NPAD, C), jnp.float32),
        mesh=mesh,
        scratch_types=[
            pltpu.VMEM((K,), jnp.int32),
            pltpu.VMEM((K,), jnp.int32),
            pltpu.VMEM((K,), jnp.float32),
            pltpu.VMEM((K, C), jnp.float32),
            pltpu.VMEM_SHARED((NPAD, C), jnp.float32),
            pltpu.SemaphoreType.DMA,
        ],
    )


def _h_body(k_ref, b_ref, h_ref):
    h_ref[...] = k_ref[...] + b_ref[...]


_SELU_SCALE = 1.0507009873554804934193349852946
_SELU_ALPHA = 1.6732632423543772848170429916717


def _post_body(p0_ref, p1_ref, sk_ref, o_ref):
    z = p0_ref[...] + p1_ref[...] + sk_ref[...]
    neg = _SELU_ALPHA * (jnp.exp(jnp.minimum(z, 0.0)) - 1.0)
    o_ref[...] = _SELU_SCALE * jnp.where(z > 0.0, z, neg)


_BLK = 1000


def kernel(x, edge_index, edge_weight, kernel, bias, skip_weight):
    del x
    src = edge_index[0].astype(jnp.int32)
    dst = edge_index[1].astype(jnp.int32)
    w = edge_weight.astype(jnp.float32)

    e = src.shape[0]
    ept = -(-e // (NW * K)) * K
    e_pad = NW * ept
    nchunk = ept // K
    if e_pad != e:
        pad = e_pad - e
        zi = jnp.zeros((pad,), jnp.int32)
        src = jnp.concatenate([src, zi])
        dst = jnp.concatenate([dst, zi])
        w = jnp.concatenate([w, jnp.zeros((pad,), jnp.float32)])

    h = pl.pallas_call(
        _h_body,
        grid=(N // _BLK,),
        in_specs=[
            pl.BlockSpec((_BLK, C), lambda i: (i, 0)),
            pl.BlockSpec((1, C), lambda i: (0, 0)),
        ],
        out_specs=pl.BlockSpec((_BLK, C), lambda i: (i, 0)),
        out_shape=jax.ShapeDtypeStruct((N, C), jnp.float32),
    )(kernel, bias.reshape(1, C))

    p = _make_sc_call(ept, nchunk)(h, src, dst, w)

    out = pl.pallas_call(
        _post_body,
        grid=(N // _BLK,),
        in_specs=[
            pl.BlockSpec((_BLK, C), lambda i: (i, 0)),
            pl.BlockSpec((_BLK, C), lambda i: (i, 0)),
            pl.BlockSpec((1, C), lambda i: (0, 0)),
        ],
        out_specs=pl.BlockSpec((_BLK, C), lambda i: (i, 0)),
        out_shape=jax.ShapeDtypeStruct((N, C), jnp.float32),
    )(p[:N], p[NPAD:NPAD + N], skip_weight.reshape(1, C))
    return out
